# Initial kernel scaffold; baseline (speedup 1.0000x reference)
import jax
import jax.numpy as jnp
from jax.experimental import pallas as pl


def kernel(x, edge_index, edge_attr, batch, emb_h, emb_e, WQ, bQ, WK, bK, WV, bV, WO, bO, Wf1, bf1, Wf2, bf2, Wm0, bm0, Wm1, bm1, Wm2, bm2):
    raise NotImplementedError("write your pallas kernel here")


# speedup vs baseline: 31.7358x; 31.7358x over previous
import functools

import jax
import jax.numpy as jnp
from jax import lax
from jax.experimental import pallas as pl
from jax.experimental.pallas import tpu as pltpu
from jax.experimental.pallas import tpu_sc as plsc

N = 10000
E = 320000
D = 128
H = 8
DK = 16
L = 4

NTILES = 32
EPT = E // NTILES
C = 40
NCH = EPT // C
ROWS = 144
NA = 10240
RPT = NA // 16
ZR = 128

BR = 400
NB = N // BR

_f32 = jnp.float32


def _embed_body(x_ref, emb_ref, o_ref):
    xv = x_ref[0]
    xb = jnp.broadcast_to(xv, (32, BR))
    oh = (xb == lax.broadcasted_iota(jnp.int32, (32, BR), 0)).astype(_f32)
    o_ref[...] = lax.dot_general(
        oh, emb_ref[...], (((0,), (0,)), ((), ())),
        preferred_element_type=_f32)


def _embed(x3, emb_pad):
    return pl.pallas_call(
        _embed_body,
        grid=(NB,),
        in_specs=[
            pl.BlockSpec((1, 1, BR), lambda i: (i, 0, 0)),
            pl.BlockSpec((32, D), lambda i: (0, 0)),
        ],
        out_specs=pl.BlockSpec((BR, D), lambda i: (i, 0)),
        out_shape=jax.ShapeDtypeStruct((N, D), _f32),
    )(x3, emb_pad)


def _qkv_body(h_ref, wq, bq, wk, bk, wv, bv, q_ref, kv_ref):
    h = h_ref[...]
    q_ref[...] = jnp.dot(h, wq[...], preferred_element_type=_f32) + bq[...]
    kv_ref[:, :D] = jnp.dot(h, wk[...], preferred_element_type=_f32) + bk[...]
    kv_ref[:, D:] = jnp.dot(h, wv[...], preferred_element_type=_f32) + bv[...]


def _qkv(h, wq, bq, wk, bk, wv, bv):
    wspec = pl.BlockSpec((D, D), lambda i: (0, 0))
    bspec = pl.BlockSpec((1, D), lambda i: (0, 0))
    return pl.pallas_call(
        _qkv_body,
        grid=(NB,),
        in_specs=[pl.BlockSpec((BR, D), lambda i: (i, 0)),
                  wspec, bspec, wspec, bspec, wspec, bspec],
        out_specs=[pl.BlockSpec((BR, D), lambda i: (i, 0)),
                   pl.BlockSpec((BR, 2 * D), lambda i: (i, 0))],
        out_shape=[jax.ShapeDtypeStruct((N, D), _f32),
                   jax.ShapeDtypeStruct((N, 2 * D), _f32)],
    )(h, wq, bq, wk, bk, wv, bv)


def _ln(h):
    mu = jnp.mean(h, axis=-1, keepdims=True)
    d = h - mu
    var = jnp.mean(d * d, axis=-1, keepdims=True)
    return d / jnp.sqrt(var + 1e-5)


def _post_body(h_ref, s0_ref, s1_ref, sel_ref, wo, bo, w1, b1, w2, b2, o_ref):
    acc = s0_ref[...] + s1_ref[...]
    msg = acc[:, :D]
    den = jnp.dot(acc, sel_ref[...], preferred_element_type=_f32)
    msgn = msg / (den + 1e-9)
    h = h_ref[...]
    h_att = jnp.dot(msgn, wo[...], preferred_element_type=_f32) + bo[...]
    h1 = _ln(h + h_att)
    f = jnp.maximum(jnp.dot(h1, w1[...], preferred_element_type=_f32) + b1[...], 0.0)
    f = jnp.dot(f, w2[...], preferred_element_type=_f32) + b2[...]
    o_ref[...] = _ln(h1 + f)


def _post(h, s0, s1, sel, wo, bo, w1, b1, w2, b2):
    return pl.pallas_call(
        _post_body,
        grid=(NB,),
        in_specs=[
            pl.BlockSpec((BR, D), lambda i: (i, 0)),
            pl.BlockSpec((BR, ROWS), lambda i: (i, 0)),
            pl.BlockSpec((BR, ROWS), lambda i: (i, 0)),
            pl.BlockSpec((ROWS, D), lambda i: (0, 0)),
            pl.BlockSpec((D, D), lambda i: (0, 0)),
            pl.BlockSpec((1, D), lambda i: (0, 0)),
            pl.BlockSpec((D, 2 * D), lambda i: (0, 0)),
            pl.BlockSpec((1, 2 * D), lambda i: (0, 0)),
            pl.BlockSpec((2 * D, D), lambda i: (0, 0)),
            pl.BlockSpec((1, D), lambda i: (0, 0)),
        ],
        out_specs=pl.BlockSpec((BR, D), lambda i: (i, 0)),
        out_shape=jax.ShapeDtypeStruct((N, D), _f32),
    )(h, s0, s1, sel, wo, bo, w1, b1, w2, b2)


def _readout_body(h_ref, w0, b0, w1, b1, w2, b2, o_ref):
    a = jnp.maximum(jnp.dot(h_ref[...], w0[...], preferred_element_type=_f32) + b0[...], 0.0)
    a = jnp.maximum(jnp.dot(a, w1[...], preferred_element_type=_f32) + b1[...], 0.0)
    o_ref[...] = jnp.dot(a, w2[...], preferred_element_type=_f32) + b2[...]


def _readout(h, w0, b0, w1, b1, w2, b2):
    wspec = pl.BlockSpec((D, D), lambda i: (0, 0))
    bspec = pl.BlockSpec((1, D), lambda i: (0, 0))
    return pl.pallas_call(
        _readout_body,
        grid=(NB,),
        in_specs=[pl.BlockSpec((BR, D), lambda i: (i, 0)),
                  wspec, bspec, wspec, bspec, wspec, bspec],
        out_specs=pl.BlockSpec((BR, D), lambda i: (i, 0)),
        out_shape=jax.ShapeDtypeStruct((N, D), _f32),
    )(h, w0, b0, w1, b1, w2, b2)


def _edge_body(q_hbm, kv_hbm, src_hbm, dst_hbm, out_hbm,
               src_c, dst_c, kv_b, q_b, ctr, acc_sh,
               sem1, sem2):
    cid = lax.axis_index("c")
    sid = lax.axis_index("s")
    wid = sid * 2 + cid

    zeros16 = jnp.zeros((16,), _f32)

    def zrow(i, _):
        for j in range(ROWS // 16):
            ctr[i, pl.ds(j * 16, 16)] = zeros16
        return _

    lax.fori_loop(0, C, zrow, 0)
    for k in range(RPT // C):
        pltpu.sync_copy(ctr, acc_sh.at[pl.ds(sid * RPT + k * C, C)])
    plsc.subcore_barrier()

    scale = 0.25
    lane = lax.iota(jnp.int32, 16)
    mask8 = lane < 8
    idx15 = jnp.full((16,), 15, jnp.int32)
    idxh = [jnp.full((16,), h, jnp.int32) for h in range(H)]

    def chunk(j, _):
        pltpu.sync_copy(src_hbm.at[wid, j], src_c)
        pltpu.sync_copy(dst_hbm.at[wid, j], dst_c)
        cp1 = pltpu.async_copy(kv_hbm.at[src_c], kv_b, sem1)
        cp2 = pltpu.async_copy(q_hbm.at[dst_c], q_b, sem2)
        cp1.wait()
        cp2.wait()

        def edge(e, _):
            sv = jnp.zeros((16,), _f32)
            for h in range(H):
                kvec = kv_b[e, pl.ds(h * DK, DK)]
                qvec = q_b[e, pl.ds(h * DK, DK)]
                c = plsc.cumsum(kvec * qvec)
                t = jnp.take_along_axis(c, idx15, axis=0, mode='promise_in_bounds')
                sv = jnp.where(lane == h, t, sv)
            ev = jnp.where(mask8, jnp.exp(jnp.minimum(sv * scale, 60.0)), 0.0)
            ctr[e, pl.ds(D, 16)] = ev
            for h in range(H):
                exv = jnp.take_along_axis(ev, idxh[h], axis=0, mode='promise_in_bounds')
                vvec = kv_b[e, pl.ds(D + h * DK, DK)]
                ctr[e, pl.ds(h * DK, DK)] = exv * vvec
            return _

        lax.fori_loop(0, C, edge, 0)
        pltpu.sync_copy(ctr, acc_sh.at[dst_c], add=True)
        return _

    lax.fori_loop(0, NCH, chunk, 0)

    plsc.subcore_barrier()
    pltpu.sync_copy(acc_sh.at[pl.ds(sid * RPT, RPT)],
                    out_hbm.at[cid, pl.ds(sid * RPT, RPT)])


def _edge(q, kv, src, dst):
    mesh = plsc.VectorSubcoreMesh(core_axis_name="c", subcore_axis_name="s")
    fn = pl.kernel(
        _edge_body,
        out_type=jax.ShapeDtypeStruct((2, NA, ROWS), _f32),
        mesh=mesh,
        scratch_types=[
            pltpu.VMEM((C,), jnp.int32),
            pltpu.VMEM((C,), jnp.int32),
            pltpu.VMEM((C, 2 * D), _f32),
            pltpu.VMEM((C, D), _f32),
            pltpu.VMEM((C, ROWS), _f32),
            pltpu.VMEM_SHARED((NA, ROWS), _f32),
            pltpu.SemaphoreType.DMA,
            pltpu.SemaphoreType.DMA,
        ],
        compiler_params=pltpu.CompilerParams(use_tc_tiling_on_sc=False, needs_layout_passes=False),
    )
    return fn(q, kv, src, dst)


def kernel(x, edge_index, edge_attr, batch, emb_h, emb_e, WQ, bQ, WK, bK,
           WV, bV, WO, bO, Wf1, bf1, Wf2, bf2, Wm0, bm0, Wm1, bm1, Wm2, bm2):
    x3 = x.astype(jnp.int32).reshape(NB, 1, BR)
    src = edge_index[0].astype(jnp.int32).reshape(NTILES, NCH, C)
    dst = edge_index[1].astype(jnp.int32).reshape(NTILES, NCH, C)

    emb_pad = jnp.zeros((32, D), _f32).at[:28].set(emb_h.astype(_f32))

    eye8 = jnp.eye(H, dtype=_f32)
    sel = jnp.zeros((ROWS, D), _f32).at[D:D + H].set(jnp.repeat(eye8, DK, axis=1))

    h = _embed(x3, emb_pad)
    for l in range(L):
        q, kv = _qkv(h, WQ[l], bQ[l].reshape(1, D), WK[l], bK[l].reshape(1, D),
                     WV[l], bV[l].reshape(1, D))
        s2 = _edge(q, kv, src, dst)
        h = _post(h, s2[0], s2[1], sel, WO[l], bO[l].reshape(1, D),
                  Wf1[l], bf1[l].reshape(1, 2 * D), Wf2[l], bf2[l].reshape(1, D))

    w0 = jnp.zeros((D, D), _f32).at[:, :D // 2].set(Wm0)
    b0 = jnp.zeros((1, D), _f32).at[0, :D // 2].set(bm0)
    w1 = jnp.zeros((D, D), _f32).at[:D // 2, :D // 4].set(Wm1)
    b1 = jnp.zeros((1, D), _f32).at[0, :D // 4].set(bm1)
    w2 = jnp.zeros((D, D), _f32).at[:D // 4, :1].set(Wm2)
    b2 = jnp.zeros((1, D), _f32).at[0, :1].set(bm2)
    o = _readout(h, w0, b0, w1, b1, w2, b2)
    return o[:, :1]


# speedup vs baseline: 40.2379x; 1.2679x over previous
import functools

import jax
import jax.numpy as jnp
from jax import lax
from jax.experimental import pallas as pl
from jax.experimental.pallas import tpu as pltpu
from jax.experimental.pallas import tpu_sc as plsc

N = 10000
E = 320000
D = 128
H = 8
DK = 16
L = 4

NTILES = 32
EPT = E // NTILES
C = 40
NCH = EPT // C
ROWS = 144
NA = 10240
RPT = NA // 16
ZR = 128

BR = 400
NB = N // BR

_f32 = jnp.float32


def _embed_body(x_ref, emb_ref, o_ref):
    xv = x_ref[0]
    xb = jnp.broadcast_to(xv, (32, BR))
    oh = (xb == lax.broadcasted_iota(jnp.int32, (32, BR), 0)).astype(_f32)
    o_ref[...] = lax.dot_general(
        oh, emb_ref[...], (((0,), (0,)), ((), ())),
        preferred_element_type=_f32)


def _embed(x3, emb_pad):
    return pl.pallas_call(
        _embed_body,
        grid=(NB,),
        in_specs=[
            pl.BlockSpec((1, 1, BR), lambda i: (i, 0, 0)),
            pl.BlockSpec((32, D), lambda i: (0, 0)),
        ],
        out_specs=pl.BlockSpec((BR, D), lambda i: (i, 0)),
        out_shape=jax.ShapeDtypeStruct((N, D), _f32),
    )(x3, emb_pad)


def _qkv_body(h_ref, wq, bq, wk, bk, wv, bv, q_ref, kv_ref):
    h = h_ref[...]
    q_ref[...] = jnp.dot(h, wq[...], preferred_element_type=_f32) + bq[...]
    kv_ref[:, :D] = jnp.dot(h, wk[...], preferred_element_type=_f32) + bk[...]
    kv_ref[:, D:] = jnp.dot(h, wv[...], preferred_element_type=_f32) + bv[...]


def _qkv(h, wq, bq, wk, bk, wv, bv):
    wspec = pl.BlockSpec((D, D), lambda i: (0, 0))
    bspec = pl.BlockSpec((1, D), lambda i: (0, 0))
    return pl.pallas_call(
        _qkv_body,
        grid=(NB,),
        in_specs=[pl.BlockSpec((BR, D), lambda i: (i, 0)),
                  wspec, bspec, wspec, bspec, wspec, bspec],
        out_specs=[pl.BlockSpec((BR, D), lambda i: (i, 0)),
                   pl.BlockSpec((BR, 2 * D), lambda i: (i, 0))],
        out_shape=[jax.ShapeDtypeStruct((N, D), _f32),
                   jax.ShapeDtypeStruct((N, 2 * D), _f32)],
    )(h, wq, bq, wk, bk, wv, bv)


def _ln(h):
    mu = jnp.mean(h, axis=-1, keepdims=True)
    d = h - mu
    var = jnp.mean(d * d, axis=-1, keepdims=True)
    return d / jnp.sqrt(var + 1e-5)


def _post_body(h_ref, s0_ref, s1_ref, sel_ref, wo, bo, w1, b1, w2, b2, o_ref):
    acc = s0_ref[...] + s1_ref[...]
    msg = acc[:, :D]
    den = jnp.dot(acc, sel_ref[...], preferred_element_type=_f32)
    msgn = msg / (den + 1e-9)
    h = h_ref[...]
    h_att = jnp.dot(msgn, wo[...], preferred_element_type=_f32) + bo[...]
    h1 = _ln(h + h_att)
    f = jnp.maximum(jnp.dot(h1, w1[...], preferred_element_type=_f32) + b1[...], 0.0)
    f = jnp.dot(f, w2[...], preferred_element_type=_f32) + b2[...]
    o_ref[...] = _ln(h1 + f)


def _post(h, s0, s1, sel, wo, bo, w1, b1, w2, b2):
    return pl.pallas_call(
        _post_body,
        grid=(NB,),
        in_specs=[
            pl.BlockSpec((BR, D), lambda i: (i, 0)),
            pl.BlockSpec((BR, ROWS), lambda i: (i, 0)),
            pl.BlockSpec((BR, ROWS), lambda i: (i, 0)),
            pl.BlockSpec((ROWS, D), lambda i: (0, 0)),
            pl.BlockSpec((D, D), lambda i: (0, 0)),
            pl.BlockSpec((1, D), lambda i: (0, 0)),
            pl.BlockSpec((D, 2 * D), lambda i: (0, 0)),
            pl.BlockSpec((1, 2 * D), lambda i: (0, 0)),
            pl.BlockSpec((2 * D, D), lambda i: (0, 0)),
            pl.BlockSpec((1, D), lambda i: (0, 0)),
        ],
        out_specs=pl.BlockSpec((BR, D), lambda i: (i, 0)),
        out_shape=jax.ShapeDtypeStruct((N, D), _f32),
    )(h, s0, s1, sel, wo, bo, w1, b1, w2, b2)


def _readout_body(h_ref, w0, b0, w1, b1, w2, b2, o_ref):
    a = jnp.maximum(jnp.dot(h_ref[...], w0[...], preferred_element_type=_f32) + b0[...], 0.0)
    a = jnp.maximum(jnp.dot(a, w1[...], preferred_element_type=_f32) + b1[...], 0.0)
    o_ref[...] = jnp.dot(a, w2[...], preferred_element_type=_f32) + b2[...]


def _readout(h, w0, b0, w1, b1, w2, b2):
    wspec = pl.BlockSpec((D, D), lambda i: (0, 0))
    bspec = pl.BlockSpec((1, D), lambda i: (0, 0))
    return pl.pallas_call(
        _readout_body,
        grid=(NB,),
        in_specs=[pl.BlockSpec((BR, D), lambda i: (i, 0)),
                  wspec, bspec, wspec, bspec, wspec, bspec],
        out_specs=pl.BlockSpec((BR, D), lambda i: (i, 0)),
        out_shape=jax.ShapeDtypeStruct((N, D), _f32),
    )(h, w0, b0, w1, b1, w2, b2)


def _edge_body(q_hbm, kv_hbm, src_hbm, dst_hbm, out_hbm,
               src_c, dst_c, kv_b, q_b, src_c2, dst_c2, kv_b2, q_b2,
               ctr, acc_sh, sem1, sem2, sem3, sem4):
    cid = lax.axis_index("c")
    sid = lax.axis_index("s")
    wid = sid * 2 + cid

    zeros16 = jnp.zeros((16,), _f32)

    def zrow(i, _):
        for j in range(ROWS // 16):
            ctr[i, pl.ds(j * 16, 16)] = zeros16
        return _

    lax.fori_loop(0, C, zrow, 0)
    for k in range(RPT // C):
        pltpu.sync_copy(ctr, acc_sh.at[pl.ds(sid * RPT + k * C, C)])
    plsc.subcore_barrier()

    scale = 0.25
    lane = lax.iota(jnp.int32, 16)
    mask8 = lane < 8
    idx15 = jnp.full((16,), 15, jnp.int32)
    idxh = [jnp.full((16,), h, jnp.int32) for h in range(H)]

    def fetch(j, sc, dc, kvb, qb, semk, semq):
        pltpu.sync_copy(src_hbm.at[wid, j], sc)
        pltpu.sync_copy(dst_hbm.at[wid, j], dc)
        pltpu.async_copy(kv_hbm.at[sc], kvb, semk)
        pltpu.async_copy(q_hbm.at[dc], qb, semq)

    def wait(sc, dc, kvb, qb, semk, semq):
        pltpu.make_async_copy(kv_hbm.at[sc], kvb, semk).wait()
        pltpu.make_async_copy(q_hbm.at[dc], qb, semq).wait()

    def compute(dc, kvb, qb):
        def edge(e, _):
            sv = jnp.zeros((16,), _f32)
            for h in range(H):
                kvec = kvb[e, pl.ds(h * DK, DK)]
                qvec = qb[e, pl.ds(h * DK, DK)]
                c = plsc.cumsum(kvec * qvec)
                t = jnp.take_along_axis(c, idx15, axis=0, mode='promise_in_bounds')
                sv = jnp.where(lane == h, t, sv)
            ev = jnp.where(mask8, jnp.exp(jnp.minimum(sv * scale, 60.0)), 0.0)
            ctr[e, pl.ds(D, 16)] = ev
            for h in range(H):
                exv = jnp.take_along_axis(ev, idxh[h], axis=0, mode='promise_in_bounds')
                vvec = kvb[e, pl.ds(D + h * DK, DK)]
                ctr[e, pl.ds(h * DK, DK)] = exv * vvec
            return _

        lax.fori_loop(0, C, edge, 0)
        pltpu.sync_copy(ctr, acc_sh.at[dc], add=True)

    bufA = (src_c, dst_c, kv_b, q_b, sem1, sem2)
    bufB = (src_c2, dst_c2, kv_b2, q_b2, sem3, sem4)

    fetch(0, *bufA)

    def pair(i, _):
        j0 = 2 * i
        fetch(j0 + 1, *bufB)
        wait(*bufA)
        compute(dst_c, kv_b, q_b)

        @pl.when(j0 + 2 < NCH)
        def _prefetch():
            fetch(j0 + 2, *bufA)

        wait(*bufB)
        compute(dst_c2, kv_b2, q_b2)
        return _

    lax.fori_loop(0, NCH // 2, pair, 0)

    plsc.subcore_barrier()
    pltpu.sync_copy(acc_sh.at[pl.ds(sid * RPT, RPT)],
                    out_hbm.at[cid, pl.ds(sid * RPT, RPT)])


def _edge(q, kv, src, dst):
    mesh = plsc.VectorSubcoreMesh(core_axis_name="c", subcore_axis_name="s")
    fn = pl.kernel(
        _edge_body,
        out_type=jax.ShapeDtypeStruct((2, NA, ROWS), _f32),
        mesh=mesh,
        scratch_types=[
            pltpu.VMEM((C,), jnp.int32),
            pltpu.VMEM((C,), jnp.int32),
            pltpu.VMEM((C, 2 * D), _f32),
            pltpu.VMEM((C, D), _f32),
            pltpu.VMEM((C,), jnp.int32),
            pltpu.VMEM((C,), jnp.int32),
            pltpu.VMEM((C, 2 * D), _f32),
            pltpu.VMEM((C, D), _f32),
            pltpu.VMEM((C, ROWS), _f32),
            pltpu.VMEM_SHARED((NA, ROWS), _f32),
            pltpu.SemaphoreType.DMA,
            pltpu.SemaphoreType.DMA,
            pltpu.SemaphoreType.DMA,
            pltpu.SemaphoreType.DMA,
        ],
        compiler_params=pltpu.CompilerParams(use_tc_tiling_on_sc=False, needs_layout_passes=False),
    )
    return fn(q, kv, src, dst)


def kernel(x, edge_index, edge_attr, batch, emb_h, emb_e, WQ, bQ, WK, bK,
           WV, bV, WO, bO, Wf1, bf1, Wf2, bf2, Wm0, bm0, Wm1, bm1, Wm2, bm2):
    x3 = x.astype(jnp.int32).reshape(NB, 1, BR)
    src = edge_index[0].astype(jnp.int32).reshape(NTILES, NCH, C)
    dst = edge_index[1].astype(jnp.int32).reshape(NTILES, NCH, C)

    emb_pad = jnp.zeros((32, D), _f32).at[:28].set(emb_h.astype(_f32))

    eye8 = jnp.eye(H, dtype=_f32)
    sel = jnp.zeros((ROWS, D), _f32).at[D:D + H].set(jnp.repeat(eye8, DK, axis=1))

    h = _embed(x3, emb_pad)
    for l in range(L):
        q, kv = _qkv(h, WQ[l], bQ[l].reshape(1, D), WK[l], bK[l].reshape(1, D),
                     WV[l], bV[l].reshape(1, D))
        s2 = _edge(q, kv, src, dst)
        h = _post(h, s2[0], s2[1], sel, WO[l], bO[l].reshape(1, D),
                  Wf1[l], bf1[l].reshape(1, 2 * D), Wf2[l], bf2[l].reshape(1, D))

    w0 = jnp.zeros((D, D), _f32).at[:, :D // 2].set(Wm0)
    b0 = jnp.zeros((1, D), _f32).at[0, :D // 2].set(bm0)
    w1 = jnp.zeros((D, D), _f32).at[:D // 2, :D // 4].set(Wm1)
    b1 = jnp.zeros((1, D), _f32).at[0, :D // 4].set(bm1)
    w2 = jnp.zeros((D, D), _f32).at[:D // 4, :1].set(Wm2)
    b2 = jnp.zeros((1, D), _f32).at[0, :1].set(bm2)
    o = _readout(h, w0, b0, w1, b1, w2, b2)
    return o[:, :1]


# speedup vs baseline: 92.4912x; 2.2986x over previous
import functools

import jax
import jax.numpy as jnp
from jax import lax
from jax.experimental import pallas as pl
from jax.experimental.pallas import tpu as pltpu
from jax.experimental.pallas import tpu_sc as plsc

N = 10000
E = 320000
D = 128
H = 8
DK = 16
L = 4

NTILES = 32
EPT = E // NTILES
C = 40
NCH = EPT // C
ROWS = 144
NA = 10240
RPT = NA // 16
ZR = 128

BR = 400
NB = N // BR

_f32 = jnp.float32


def _embed_body(x_ref, emb_ref, o_ref):
    xv = x_ref[0]
    xb = jnp.broadcast_to(xv, (32, BR))
    oh = (xb == lax.broadcasted_iota(jnp.int32, (32, BR), 0)).astype(_f32)
    o_ref[...] = lax.dot_general(
        oh, emb_ref[...], (((0,), (0,)), ((), ())),
        preferred_element_type=_f32)


def _embed(x3, emb_pad):
    return pl.pallas_call(
        _embed_body,
        grid=(NB,),
        in_specs=[
            pl.BlockSpec((1, 1, BR), lambda i: (i, 0, 0)),
            pl.BlockSpec((32, D), lambda i: (0, 0)),
        ],
        out_specs=pl.BlockSpec((BR, D), lambda i: (i, 0)),
        out_shape=jax.ShapeDtypeStruct((N, D), _f32),
    )(x3, emb_pad)


def _qkv_body(h_ref, wq, bq, wk, bk, wv, bv, q_ref, kv_ref):
    h = h_ref[...]
    q_ref[...] = jnp.dot(h, wq[...], preferred_element_type=_f32) + bq[...]
    kv_ref[:, :D] = jnp.dot(h, wk[...], preferred_element_type=_f32) + bk[...]
    kv_ref[:, D:] = jnp.dot(h, wv[...], preferred_element_type=_f32) + bv[...]


def _qkv(h, wq, bq, wk, bk, wv, bv):
    wspec = pl.BlockSpec((D, D), lambda i: (0, 0))
    bspec = pl.BlockSpec((1, D), lambda i: (0, 0))
    return pl.pallas_call(
        _qkv_body,
        grid=(NB,),
        in_specs=[pl.BlockSpec((BR, D), lambda i: (i, 0)),
                  wspec, bspec, wspec, bspec, wspec, bspec],
        out_specs=[pl.BlockSpec((BR, D), lambda i: (i, 0)),
                   pl.BlockSpec((BR, 2 * D), lambda i: (i, 0))],
        out_shape=[jax.ShapeDtypeStruct((N, D), _f32),
                   jax.ShapeDtypeStruct((N, 2 * D), _f32)],
    )(h, wq, bq, wk, bk, wv, bv)


def _ln(h):
    mu = jnp.mean(h, axis=-1, keepdims=True)
    d = h - mu
    var = jnp.mean(d * d, axis=-1, keepdims=True)
    return d / jnp.sqrt(var + 1e-5)


def _post_body(h_ref, s0_ref, s1_ref, sel_ref, wo, bo, w1, b1, w2, b2, o_ref):
    acc = s0_ref[...] + s1_ref[...]
    msg = acc[:, :D]
    den = jnp.dot(acc, sel_ref[...], preferred_element_type=_f32)
    msgn = msg / (den + 1e-9)
    h = h_ref[...]
    h_att = jnp.dot(msgn, wo[...], preferred_element_type=_f32) + bo[...]
    h1 = _ln(h + h_att)
    f = jnp.maximum(jnp.dot(h1, w1[...], preferred_element_type=_f32) + b1[...], 0.0)
    f = jnp.dot(f, w2[...], preferred_element_type=_f32) + b2[...]
    o_ref[...] = _ln(h1 + f)


def _post(h, s0, s1, sel, wo, bo, w1, b1, w2, b2):
    return pl.pallas_call(
        _post_body,
        grid=(NB,),
        in_specs=[
            pl.BlockSpec((BR, D), lambda i: (i, 0)),
            pl.BlockSpec((BR, ROWS), lambda i: (i, 0)),
            pl.BlockSpec((BR, ROWS), lambda i: (i, 0)),
            pl.BlockSpec((ROWS, D), lambda i: (0, 0)),
            pl.BlockSpec((D, D), lambda i: (0, 0)),
            pl.BlockSpec((1, D), lambda i: (0, 0)),
            pl.BlockSpec((D, 2 * D), lambda i: (0, 0)),
            pl.BlockSpec((1, 2 * D), lambda i: (0, 0)),
            pl.BlockSpec((2 * D, D), lambda i: (0, 0)),
            pl.BlockSpec((1, D), lambda i: (0, 0)),
        ],
        out_specs=pl.BlockSpec((BR, D), lambda i: (i, 0)),
        out_shape=jax.ShapeDtypeStruct((N, D), _f32),
    )(h, s0, s1, sel, wo, bo, w1, b1, w2, b2)


def _readout_body(h_ref, w0, b0, w1, b1, w2, b2, o_ref):
    a = jnp.maximum(jnp.dot(h_ref[...], w0[...], preferred_element_type=_f32) + b0[...], 0.0)
    a = jnp.maximum(jnp.dot(a, w1[...], preferred_element_type=_f32) + b1[...], 0.0)
    o_ref[...] = jnp.dot(a, w2[...], preferred_element_type=_f32) + b2[...]


def _readout(h, w0, b0, w1, b1, w2, b2):
    wspec = pl.BlockSpec((D, D), lambda i: (0, 0))
    bspec = pl.BlockSpec((1, D), lambda i: (0, 0))
    return pl.pallas_call(
        _readout_body,
        grid=(NB,),
        in_specs=[pl.BlockSpec((BR, D), lambda i: (i, 0)),
                  wspec, bspec, wspec, bspec, wspec, bspec],
        out_specs=pl.BlockSpec((BR, D), lambda i: (i, 0)),
        out_shape=jax.ShapeDtypeStruct((N, D), _f32),
    )(h, w0, b0, w1, b1, w2, b2)


def _edge_body(q_hbm, kv_hbm, src_hbm, dst_hbm, out_hbm,
               src_c, dst_c, kv_b, q_b, src_c2, dst_c2, kv_b2, q_b2,
               ctr, acc_sh, sem1, sem2, sem3, sem4,
               semi1, semi2, semi3, semi4):
    cid = lax.axis_index("c")
    sid = lax.axis_index("s")
    wid = sid * 2 + cid

    zeros16 = jnp.zeros((16,), _f32)

    def zrow(i, _):
        for j in range(ROWS // 16):
            ctr[i, pl.ds(j * 16, 16)] = zeros16
        return _

    lax.fori_loop(0, C, zrow, 0)
    for k in range(RPT // C):
        pltpu.sync_copy(ctr, acc_sh.at[pl.ds(sid * RPT + k * C, C)])
    plsc.subcore_barrier()

    scale = 0.25
    lane = lax.iota(jnp.int32, 16)
    mask8 = lane < 8
    idx15 = jnp.full((16,), 15, jnp.int32)
    idxh = [jnp.full((16,), h, jnp.int32) for h in range(H)]

    def fetch_src(j, sc, semi):
        pltpu.async_copy(src_hbm.at[wid, j], sc, semi)

    def fetch_dst(j, dc, semi):
        pltpu.async_copy(dst_hbm.at[wid, j], dc, semi)

    def wait_idx(j, buf, semi, hbm):
        pltpu.make_async_copy(hbm.at[wid, j], buf, semi).wait()

    def gather(sc, dc, kvb, qb, semk, semq):
        pltpu.async_copy(kv_hbm.at[sc], kvb, semk)
        pltpu.async_copy(q_hbm.at[dc], qb, semq)

    def wait_gather(sc, dc, kvb, qb, semk, semq):
        pltpu.make_async_copy(kv_hbm.at[sc], kvb, semk).wait()
        pltpu.make_async_copy(q_hbm.at[dc], qb, semq).wait()

    def compute(dc, kvb, qb):
        @plsc.parallel_loop(0, C, 1, unroll=2)
        def edge(e):
            sv = jnp.zeros((16,), _f32)
            for h in range(H):
                kvec = kvb[e, pl.ds(h * DK, DK)]
                qvec = qb[e, pl.ds(h * DK, DK)]
                c = plsc.cumsum(kvec * qvec)
                t = jnp.take_along_axis(c, idx15, axis=0, mode='promise_in_bounds')
                sv = jnp.where(lane == h, t, sv)
            ev = jnp.where(mask8, jnp.exp(jnp.minimum(sv * scale, 60.0)), 0.0)
            ctr[e, pl.ds(D, 16)] = ev
            for h in range(H):
                exv = jnp.take_along_axis(ev, idxh[h], axis=0, mode='promise_in_bounds')
                vvec = kvb[e, pl.ds(D + h * DK, DK)]
                ctr[e, pl.ds(h * DK, DK)] = exv * vvec

        pltpu.sync_copy(ctr, acc_sh.at[dc], add=True)

    gsA = (src_c, dst_c, kv_b, q_b, sem1, sem2)
    gsB = (src_c2, dst_c2, kv_b2, q_b2, sem3, sem4)

    fetch_src(0, src_c, semi1)
    fetch_dst(0, dst_c, semi2)
    fetch_src(1, src_c2, semi3)
    fetch_dst(1, dst_c2, semi4)
    wait_idx(0, src_c, semi1, src_hbm)
    wait_idx(0, dst_c, semi2, dst_hbm)
    gather(*gsA)

    def pair(i, _):
        j0 = 2 * i
        wait_idx(j0 + 1, src_c2, semi3, src_hbm)
        wait_idx(j0 + 1, dst_c2, semi4, dst_hbm)
        gather(*gsB)

        wait_gather(*gsA)

        @pl.when(j0 + 2 < NCH)
        def _pf_src_a():
            fetch_src(j0 + 2, src_c, semi1)

        compute(dst_c, kv_b, q_b)

        @pl.when(j0 + 2 < NCH)
        def _next_a():
            fetch_dst(j0 + 2, dst_c, semi2)
            wait_idx(j0 + 2, src_c, semi1, src_hbm)
            wait_idx(j0 + 2, dst_c, semi2, dst_hbm)
            gather(*gsA)

        wait_gather(*gsB)

        @pl.when(j0 + 3 < NCH)
        def _pf_src_b():
            fetch_src(j0 + 3, src_c2, semi3)

        compute(dst_c2, kv_b2, q_b2)

        @pl.when(j0 + 3 < NCH)
        def _pf_dst_b():
            fetch_dst(j0 + 3, dst_c2, semi4)

        return _

    lax.fori_loop(0, NCH // 2, pair, 0)

    plsc.subcore_barrier()
    pltpu.sync_copy(acc_sh.at[pl.ds(sid * RPT, RPT)],
                    out_hbm.at[cid, pl.ds(sid * RPT, RPT)])


def _edge(q, kv, src, dst):
    mesh = plsc.VectorSubcoreMesh(core_axis_name="c", subcore_axis_name="s")
    fn = pl.kernel(
        _edge_body,
        out_type=jax.ShapeDtypeStruct((2, NA, ROWS), _f32),
        mesh=mesh,
        scratch_types=[
            pltpu.VMEM((C,), jnp.int32),
            pltpu.VMEM((C,), jnp.int32),
            pltpu.VMEM((C, 2 * D), _f32),
            pltpu.VMEM((C, D), _f32),
            pltpu.VMEM((C,), jnp.int32),
            pltpu.VMEM((C,), jnp.int32),
            pltpu.VMEM((C, 2 * D), _f32),
            pltpu.VMEM((C, D), _f32),
            pltpu.VMEM((C, ROWS), _f32),
            pltpu.VMEM_SHARED((NA, ROWS), _f32),
            pltpu.SemaphoreType.DMA,
            pltpu.SemaphoreType.DMA,
            pltpu.SemaphoreType.DMA,
            pltpu.SemaphoreType.DMA,
            pltpu.SemaphoreType.DMA,
            pltpu.SemaphoreType.DMA,
            pltpu.SemaphoreType.DMA,
            pltpu.SemaphoreType.DMA,
        ],
        compiler_params=pltpu.CompilerParams(use_tc_tiling_on_sc=False, needs_layout_passes=False),
    )
    return fn(q, kv, src, dst)


def kernel(x, edge_index, edge_attr, batch, emb_h, emb_e, WQ, bQ, WK, bK,
           WV, bV, WO, bO, Wf1, bf1, Wf2, bf2, Wm0, bm0, Wm1, bm1, Wm2, bm2):
    x3 = x.astype(jnp.int32).reshape(NB, 1, BR)
    src = edge_index[0].astype(jnp.int32).reshape(NTILES, NCH, C)
    dst = edge_index[1].astype(jnp.int32).reshape(NTILES, NCH, C)

    emb_pad = jnp.zeros((32, D), _f32).at[:28].set(emb_h.astype(_f32))

    eye8 = jnp.eye(H, dtype=_f32)
    sel = jnp.zeros((ROWS, D), _f32).at[D:D + H].set(jnp.repeat(eye8, DK, axis=1))

    h = _embed(x3, emb_pad)
    for l in range(L):
        q, kv = _qkv(h, WQ[l], bQ[l].reshape(1, D), WK[l], bK[l].reshape(1, D),
                     WV[l], bV[l].reshape(1, D))
        s2 = _edge(q, kv, src, dst)
        h = _post(h, s2[0], s2[1], sel, WO[l], bO[l].reshape(1, D),
                  Wf1[l], bf1[l].reshape(1, 2 * D), Wf2[l], bf2[l].reshape(1, D))

    w0 = jnp.zeros((D, D), _f32).at[:, :D // 2].set(Wm0)
    b0 = jnp.zeros((1, D), _f32).at[0, :D // 2].set(bm0)
    w1 = jnp.zeros((D, D), _f32).at[:D // 2, :D // 4].set(Wm1)
    b1 = jnp.zeros((1, D), _f32).at[0, :D // 4].set(bm1)
    w2 = jnp.zeros((D, D), _f32).at[:D // 4, :1].set(Wm2)
    b2 = jnp.zeros((1, D), _f32).at[0, :1].set(bm2)
    o = _readout(h, w0, b0, w1, b1, w2, b2)
    return o[:, :1]
